# baseline (speedup 1.0000x reference)
import jax
import jax.numpy as jnp
from jax.experimental import pallas as pl
from jax.experimental.pallas import tpu as pltpu

_BK = 2048
_NBUF = 3


def _body(pos_ref, x_hbm, o_hbm, xbuf, obuf, insem, outsem):
    P, B, D = x_hbm.shape
    per_plane = B // _BK
    nchunks = P * per_plane

    def in_copy(chunk, slot):
        p, i = divmod(chunk, per_plane)
        return pltpu.make_async_copy(
            x_hbm.at[p, pl.ds(i * _BK, _BK)], xbuf.at[slot], insem.at[slot])

    def out_copy(chunk, slot):
        p, i = divmod(chunk, per_plane)
        return pltpu.make_async_copy(
            obuf.at[slot], o_hbm.at[p, pl.ds(i * _BK, _BK)], outsem.at[slot])

    for k in range(min(_NBUF, nchunks)):
        in_copy(k, k).start()

    for c in range(nchunks):
        slot = c % _NBUF
        p = c // per_plane
        in_copy(c, slot).wait()
        if c >= _NBUF:
            out_copy(c - _NBUF, slot).wait()
        obuf[slot] = xbuf[slot] + pos_ref[pl.ds(p, 1)]
        out_copy(c, slot).start()
        if c + _NBUF < nchunks:
            in_copy(c + _NBUF, slot).start()

    for c in range(max(0, nchunks - _NBUF), nchunks):
        out_copy(c, c % _NBUF).wait()


def kernel(patch, pos_emb):
    B, P, D = patch.shape
    xt = jnp.transpose(patch, (1, 0, 2))
    out = pl.pallas_call(
        _body,
        in_specs=[
            pl.BlockSpec((P, D), lambda: (0, 0)),
            pl.BlockSpec(memory_space=pl.ANY),
        ],
        out_specs=pl.BlockSpec(memory_space=pl.ANY),
        out_shape=jax.ShapeDtypeStruct((P, B, D), patch.dtype),
        scratch_shapes=[
            pltpu.VMEM((_NBUF, _BK, D), patch.dtype),
            pltpu.VMEM((_NBUF, _BK, D), patch.dtype),
            pltpu.SemaphoreType.DMA((_NBUF,)),
            pltpu.SemaphoreType.DMA((_NBUF,)),
        ],
    )(pos_emb, xt)
    return jnp.transpose(out, (1, 0, 2))


# speedup vs baseline: 1.0058x; 1.0058x over previous
import jax
import jax.numpy as jnp
from jax.experimental import pallas as pl
from jax.experimental.pallas import tpu as pltpu

_BK = 2048
_NBUF = 3


def _body(pos_hbm, x_hbm, o_hbm, pos_ref, xbuf, obuf, possem, insem, outsem):
    P, B, D = x_hbm.shape
    per_plane = B // _BK
    nchunks = P * per_plane

    def in_copy(chunk, slot):
        p, i = divmod(chunk, per_plane)
        return pltpu.make_async_copy(
            x_hbm.at[p, pl.ds(i * _BK, _BK)], xbuf.at[slot], insem.at[slot])

    def out_copy(chunk, slot):
        p, i = divmod(chunk, per_plane)
        return pltpu.make_async_copy(
            obuf.at[slot], o_hbm.at[p, pl.ds(i * _BK, _BK)], outsem.at[slot])

    pos_copy = pltpu.make_async_copy(pos_hbm, pos_ref, possem)
    pos_copy.start()
    for k in range(min(_NBUF, nchunks)):
        in_copy(k, k).start()
    pos_copy.wait()

    for c in range(nchunks):
        slot = c % _NBUF
        p = c // per_plane
        in_copy(c, slot).wait()
        if c >= _NBUF:
            out_copy(c - _NBUF, slot).wait()
        obuf[slot] = xbuf[slot] + pos_ref[pl.ds(p, 1)]
        out_copy(c, slot).start()
        if c + _NBUF < nchunks:
            in_copy(c + _NBUF, slot).start()

    for c in range(max(0, nchunks - _NBUF), nchunks):
        out_copy(c, c % _NBUF).wait()


def kernel(patch, pos_emb):
    B, P, D = patch.shape
    xt = jnp.transpose(patch, (1, 0, 2))
    out = pl.pallas_call(
        _body,
        in_specs=[
            pl.BlockSpec(memory_space=pl.ANY),
            pl.BlockSpec(memory_space=pl.ANY),
        ],
        out_specs=pl.BlockSpec(memory_space=pl.ANY),
        out_shape=jax.ShapeDtypeStruct((P, B, D), patch.dtype),
        scratch_shapes=[
            pltpu.VMEM((P, D), patch.dtype),
            pltpu.VMEM((_NBUF, _BK, D), patch.dtype),
            pltpu.VMEM((_NBUF, _BK, D), patch.dtype),
            pltpu.SemaphoreType.DMA,
            pltpu.SemaphoreType.DMA((_NBUF,)),
            pltpu.SemaphoreType.DMA((_NBUF,)),
        ],
    )(pos_emb, xt)
    return jnp.transpose(out, (1, 0, 2))
